# baseline (speedup 1.0000x reference)
import functools

import jax
import jax.numpy as jnp
from jax import lax
from jax.experimental import pallas as pl
from jax.experimental.pallas import tpu as pltpu
from jax.experimental.pallas import tpu_sc as plsc

NUM_FACTORS = 32
BATCH = 16384
NC, NS = 2, 16
NW = NC * NS
BPW = BATCH // NW
L = 16
TW = 128


def _gmf_sc(users, items, ut_t, it_t, w_flat, b_vec):
    mesh = plsc.VectorSubcoreMesh(core_axis_name="c", subcore_axis_name="s")

    @functools.partial(
        pl.kernel,
        mesh=mesh,
        compiler_params=pltpu.CompilerParams(needs_layout_passes=False),
        out_type=jax.ShapeDtypeStruct((BATCH,), jnp.float32),
        scratch_types=[
            pltpu.VMEM((BPW,), jnp.int32),
            pltpu.VMEM((BPW,), jnp.int32),
            pltpu.VMEM((NUM_FACTORS,), jnp.float32),
            pltpu.VMEM((L,), jnp.float32),
            pltpu.VMEM((L + 1, NUM_FACTORS, TW), jnp.float32),
            pltpu.VMEM((NUM_FACTORS, L), jnp.float32),
            pltpu.VMEM((NUM_FACTORS, L), jnp.float32),
            pltpu.VMEM((BPW,), jnp.float32),
            pltpu.SemaphoreType.DMA,
        ],
    )
    def k(u_hbm, i_hbm, ut_hbm, it_hbm, w_hbm, b_hbm, o_hbm,
          uidx_v, iidx_v, wv, bv, blk, uext, iext, acc, sem):
        wid = lax.axis_index("s") * NC + lax.axis_index("c")
        base = wid * BPW
        pltpu.sync_copy(u_hbm.at[pl.ds(base, BPW)], uidx_v)
        pltpu.sync_copy(i_hbm.at[pl.ds(base, BPW)], iidx_v)
        pltpu.sync_copy(w_hbm, wv)
        pltpu.sync_copy(b_hbm, bv)

        bias = bv[pl.ds(0, L)]
        w0 = wv[pl.ds(0, L)]
        w1 = wv[pl.ds(L, L)]
        rows = lax.iota(jnp.int32, L)

        def fetch_and_extract(tab_hbm, idx_v, ext, c0):
            idx_vec = idx_v[pl.ds(c0, L)]
            for jc in range(L):
                uo = pl.multiple_of(idx_vec[jc] & ~(TW - 1), TW)
                pltpu.async_copy(tab_hbm.at[:, pl.ds(uo, TW)], blk.at[jc], sem)
                do = pl.multiple_of((idx_vec[jc] >> 1) & ~(TW - 1), TW)
                pltpu.async_copy(tab_hbm.at[:, pl.ds(do, TW)], blk.at[L], sem)

            @pl.loop(0, 2 * L)
            def _(jc):
                pltpu.make_async_copy(
                    tab_hbm.at[:, pl.ds(0, TW)], blk.at[0], sem).wait()

            lanes = idx_vec & (TW - 1)
            for f in range(NUM_FACTORS):
                col = jnp.full((L,), f, jnp.int32)
                ext[f, :] = plsc.load_gather(blk, [rows, col, lanes])

        @pl.loop(0, BPW, step=L)
        def _(c0):
            fetch_and_extract(ut_hbm, uidx_v, uext, c0)
            fetch_and_extract(it_hbm, iidx_v, iext, c0)

            accv = jnp.zeros((L,), jnp.float32)
            for f in range(NUM_FACTORS):
                wf = w0[f] if f < L else w1[f - L]
                accv = accv + uext[f, :] * iext[f, :] * wf
            acc[pl.ds(c0, L)] = 1.0 / (1.0 + jnp.exp(-(accv + bias)))

        pltpu.sync_copy(acc, o_hbm.at[pl.ds(base, BPW)])

    return k(users, items, ut_t, it_t, w_flat, b_vec)


def kernel(users, items, user_table, item_table, W, b):
    users = users.astype(jnp.int32)
    items = items.astype(jnp.int32)
    b_vec = jnp.broadcast_to(b.astype(jnp.float32), (L,))
    return _gmf_sc(users, items, user_table.T, item_table.T,
                   W.reshape(-1), b_vec)


# speedup vs baseline: 1.6212x; 1.6212x over previous
import functools

import jax
import jax.numpy as jnp
from jax import lax
from jax.experimental import pallas as pl
from jax.experimental.pallas import tpu as pltpu
from jax.experimental.pallas import tpu_sc as plsc

NUM_FACTORS = 32
BATCH = 16384
NC, NS = 2, 16
NW = NC * NS
BPW = BATCH // NW
L = 16
TW = 128
NSLOT = 24


def _gmf_sc(users, items, ut_t, it_t, w_flat, b_vec):
    mesh = plsc.VectorSubcoreMesh(core_axis_name="c", subcore_axis_name="s")

    @functools.partial(
        pl.kernel,
        mesh=mesh,
        compiler_params=pltpu.CompilerParams(needs_layout_passes=False),
        out_type=jax.ShapeDtypeStruct((BATCH,), jnp.float32),
        scratch_types=[
            pltpu.VMEM((BPW,), jnp.int32),
            pltpu.VMEM((BPW,), jnp.int32),
            pltpu.VMEM((NUM_FACTORS,), jnp.float32),
            pltpu.VMEM((L,), jnp.float32),
            pltpu.VMEM((NSLOT, NUM_FACTORS, TW), jnp.float32),
            pltpu.VMEM((NUM_FACTORS, L), jnp.float32),
            pltpu.VMEM((NUM_FACTORS, L), jnp.float32),
            pltpu.VMEM((BPW,), jnp.float32),
            pltpu.SemaphoreType.DMA,
            pltpu.SemaphoreType.DMA,
            pltpu.SemaphoreType.DMA,
            pltpu.SemaphoreType.DMA,
        ],
    )
    def k(u_hbm, i_hbm, ut_hbm, it_hbm, w_hbm, b_hbm, o_hbm,
          uidx_v, iidx_v, wv, bv, blk, uext, iext, acc,
          sem_u1, sem_u2, sem_i1, sem_i2):
        wid = lax.axis_index("s") * NC + lax.axis_index("c")
        base = wid * BPW
        pltpu.sync_copy(u_hbm.at[pl.ds(base, BPW)], uidx_v)
        pltpu.sync_copy(i_hbm.at[pl.ds(base, BPW)], iidx_v)
        pltpu.sync_copy(w_hbm, wv)
        pltpu.sync_copy(b_hbm, bv)

        bias = bv[pl.ds(0, L)]
        w0 = wv[pl.ds(0, L)]
        w1 = wv[pl.ds(L, L)]
        rows = lax.iota(jnp.int32, L)

        def fire(tab_hbm, idx_v, c0, half, slot_base, sem):
            idx_vec = idx_v[pl.ds(c0, L)]
            for jj in range(8):
                uo = pl.multiple_of(idx_vec[8 * half + jj] & ~(TW - 1), TW)
                pltpu.async_copy(
                    tab_hbm.at[:, pl.ds(uo, TW)], blk.at[slot_base + jj], sem)

        def drain(sem):
            @pl.loop(0, 8)
            def _(jj):
                pltpu.make_async_copy(
                    ut_hbm.at[:, pl.ds(0, TW)], blk.at[0], sem).wait()

        def extract(idx_v, ext, c0, b_lo, b_hi):
            idx_vec = idx_v[pl.ds(c0, L)]
            lanes = idx_vec & (TW - 1)
            d0 = jnp.where(rows < 8, b_lo + rows, b_hi + rows - 8)
            for f in range(NUM_FACTORS):
                col = jnp.full((L,), f, jnp.int32)
                ext[f, :] = plsc.load_gather(blk, [d0, col, lanes])

        fire(ut_hbm, uidx_v, 0, 0, 0, sem_u1)
        fire(ut_hbm, uidx_v, 0, 1, 8, sem_u2)
        fire(it_hbm, iidx_v, 0, 0, 16, sem_i1)

        @pl.loop(0, BPW, step=L)
        def _(c0):
            m = lax.rem(c0 >> 4, 3)
            b0 = 8 * m
            b1 = 8 * lax.rem(m + 1, 3)
            b2 = 8 * lax.rem(m + 2, 3)
            c0n = c0 + L

            drain(sem_u1)
            drain(sem_u2)
            extract(uidx_v, uext, c0, b0, b1)

            @pl.when(c0n < BPW)
            def _():
                fire(it_hbm, iidx_v, c0, 1, b0, sem_i2)
                fire(ut_hbm, uidx_v, c0n, 0, b1, sem_u1)

            @pl.when(c0n >= BPW)
            def _():
                fire(it_hbm, iidx_v, c0, 1, b0, sem_i2)

            drain(sem_i1)
            drain(sem_i2)
            extract(iidx_v, iext, c0, b2, b0)

            @pl.when(c0n < BPW)
            def _():
                fire(ut_hbm, uidx_v, c0n, 1, b2, sem_u2)
                fire(it_hbm, iidx_v, c0n, 0, b0, sem_i1)

            accv = jnp.zeros((L,), jnp.float32)
            for f in range(NUM_FACTORS):
                wf = w0[f] if f < L else w1[f - L]
                accv = accv + uext[f, :] * iext[f, :] * wf
            acc[pl.ds(c0, L)] = 1.0 / (1.0 + jnp.exp(-(accv + bias)))

        pltpu.sync_copy(acc, o_hbm.at[pl.ds(base, BPW)])

    return k(users, items, ut_t, it_t, w_flat, b_vec)


def kernel(users, items, user_table, item_table, W, b):
    users = users.astype(jnp.int32)
    items = items.astype(jnp.int32)
    b_vec = jnp.broadcast_to(b.astype(jnp.float32), (L,))
    return _gmf_sc(users, items, user_table.T, item_table.T,
                   W.reshape(-1), b_vec)


# speedup vs baseline: 1.6773x; 1.0346x over previous
import functools

import jax
import jax.numpy as jnp
from jax import lax
from jax.experimental import pallas as pl
from jax.experimental.pallas import tpu as pltpu
from jax.experimental.pallas import tpu_sc as plsc

NUM_FACTORS = 32
BATCH = 16384
NC, NS = 2, 16
NW = NC * NS
BPW = BATCH // NW
L = 16
TW = 128


def _gmf_sc(users, items, ut_t, it_t, w_flat, b_vec):
    mesh = plsc.VectorSubcoreMesh(core_axis_name="c", subcore_axis_name="s")

    @functools.partial(
        pl.kernel,
        mesh=mesh,
        compiler_params=pltpu.CompilerParams(needs_layout_passes=False),
        out_type=jax.ShapeDtypeStruct((BATCH,), jnp.float32),
        scratch_types=[
            pltpu.VMEM((BPW,), jnp.int32),
            pltpu.VMEM((BPW,), jnp.int32),
            pltpu.VMEM((NUM_FACTORS,), jnp.float32),
            pltpu.VMEM((L,), jnp.float32),
            pltpu.VMEM((L, NUM_FACTORS, TW), jnp.float32),
            pltpu.VMEM((NUM_FACTORS, L), jnp.float32),
            pltpu.VMEM((NUM_FACTORS, L), jnp.float32),
            pltpu.VMEM((BPW,), jnp.float32),
            pltpu.SemaphoreType.DMA,
        ],
    )
    def k(u_hbm, i_hbm, ut_hbm, it_hbm, w_hbm, b_hbm, o_hbm,
          uidx_v, iidx_v, wv, bv, blk, uext, iext, acc, sem):
        wid = lax.axis_index("s") * NC + lax.axis_index("c")
        base = wid * BPW
        pltpu.sync_copy(u_hbm.at[pl.ds(base, BPW)], uidx_v)
        pltpu.sync_copy(i_hbm.at[pl.ds(base, BPW)], iidx_v)
        pltpu.sync_copy(w_hbm, wv)
        pltpu.sync_copy(b_hbm, bv)

        bias = bv[pl.ds(0, L)]
        w0 = wv[pl.ds(0, L)]
        w1 = wv[pl.ds(L, L)]
        rows = lax.iota(jnp.int32, L)

        def fetch_and_extract(tab_hbm, idx_v, ext, c0):
            idx_vec = idx_v[pl.ds(c0, L)]
            for jc in range(L):
                uo = pl.multiple_of(idx_vec[jc] & ~(TW - 1), TW)
                pltpu.async_copy(tab_hbm.at[:, pl.ds(uo, TW)], blk.at[jc], sem)

            @pl.loop(0, L)
            def _(jc):
                pltpu.make_async_copy(
                    tab_hbm.at[:, pl.ds(0, TW)], blk.at[jc], sem).wait()

            lanes = idx_vec & (TW - 1)
            for f in range(NUM_FACTORS):
                col = jnp.full((L,), f, jnp.int32)
                ext[f, :] = plsc.load_gather(blk, [rows, col, lanes])

        @pl.loop(0, BPW, step=L)
        def _(c0):
            fetch_and_extract(ut_hbm, uidx_v, uext, c0)
            fetch_and_extract(it_hbm, iidx_v, iext, c0)

            accv = jnp.zeros((L,), jnp.float32)
            for f in range(NUM_FACTORS):
                wf = w0[f] if f < L else w1[f - L]
                accv = accv + uext[f, :] * iext[f, :] * wf
            acc[pl.ds(c0, L)] = 1.0 / (1.0 + jnp.exp(-(accv + bias)))

        pltpu.sync_copy(acc, o_hbm.at[pl.ds(base, BPW)])

    return k(users, items, ut_t, it_t, w_flat, b_vec)


def kernel(users, items, user_table, item_table, W, b):
    users = users.astype(jnp.int32)
    items = items.astype(jnp.int32)
    b_vec = jnp.broadcast_to(b.astype(jnp.float32), (L,))
    return _gmf_sc(users, items, user_table.T, item_table.T,
                   W.reshape(-1), b_vec)
